# Initial kernel scaffold; baseline (speedup 1.0000x reference)
import functools

import jax
import jax.numpy as jnp
from jax import lax
from jax.experimental import pallas as pl
from jax.experimental.pallas import tpu as pltpu
from jax.experimental.pallas import tpu_sc as plsc

NU, NI = 20000, 30000
NN = NU + NI
NE = 800000
LAT = 64
Q = 32
RB = 1000
NC, NS = 2, 16
EPT = NE // NS
EC = 1000
RPS = NN // NS
ZR = 625


def _leaky(x):
  return jnp.where(x > 0, x, 0.01 * x)


def _l2n(x):
  n = jnp.sqrt(jnp.sum(x * x, axis=-1, keepdims=True))
  return x / jnp.maximum(n, 1e-12)


def _embed_body(t_ref, im_ref, id_ref, w1_ref, b1_ref, w2_ref, b2_ref,
                wi1_ref, bi1_ref, wi2_ref, bi2_ref, o_ref):
  hp = lax.Precision.HIGHEST
  h = lax.dot_general(t_ref[...], w1_ref[...], (((1,), (0,)), ((), ())),
                      precision=hp) + b1_ref[...]
  et = lax.dot_general(_leaky(h), w2_ref[...], (((1,), (0,)), ((), ())),
                       precision=hp) + b2_ref[...]
  hi = lax.dot_general(im_ref[...], wi1_ref[...], (((1,), (0,)), ((), ())),
                       precision=hp) + bi1_ref[...]
  ei = lax.dot_general(_leaky(hi), wi2_ref[...], (((1,), (0,)), ((), ())),
                       precision=hp) + bi2_ref[...]
  mm = _l2n(et) + _l2n(ei)
  o_ref[0] = mm[:, :Q]
  o_ref[1] = mm[:, Q:]
  o_ref[2] = id_ref[:, :Q]
  o_ref[3] = id_ref[:, Q:]


def _spmm_body(x_hbm, er_hbm, ec_hbm, ew_hbm, y_hbm,
               colv, rowv, wv, rows, zbuf, acc, sem):
  c = lax.axis_index("c")
  s = lax.axis_index("s")
  ebase = s * EPT
  rbase = s * RPS

  def zb(j, _):
    zbuf[j, pl.ds(0, 16)] = jnp.zeros((16,), jnp.float32)
    zbuf[j, pl.ds(16, 16)] = jnp.zeros((16,), jnp.float32)
    return 0
  lax.fori_loop(0, ZR, zb, 0)

  for p in range(2):
    q = c + 2 * p
    for z in range(RPS // ZR):
      pltpu.sync_copy(zbuf, acc.at[pl.ds(rbase + z * ZR, ZR)])
    plsc.subcore_barrier()

    def chunk(i, _):
      base = ebase + i * EC
      pltpu.sync_copy(ec_hbm.at[pl.ds(base, EC)], colv)
      pltpu.sync_copy(er_hbm.at[pl.ds(base, EC)], rowv)
      pltpu.sync_copy(ew_hbm.at[pl.ds(base, EC)], wv)
      pltpu.async_copy(x_hbm.at[q].at[colv], rows, sem).wait()

      def scale(j, _):
        ws = jnp.full((16,), wv[j], jnp.float32)
        rows[j, pl.ds(0, 16)] = rows[j, pl.ds(0, 16)] * ws
        rows[j, pl.ds(16, 16)] = rows[j, pl.ds(16, 16)] * ws
        return 0
      lax.fori_loop(0, EC, scale, 0, unroll=8)

      pltpu.sync_copy(rows, acc.at[rowv], add=True)
      return 0
    lax.fori_loop(0, EPT // EC, chunk, 0)
    plsc.subcore_barrier()

    for z in range(RPS // ZR):
      off = rbase + z * ZR
      pltpu.sync_copy(acc.at[pl.ds(off, ZR)], y_hbm.at[q].at[pl.ds(off, ZR)])
    plsc.subcore_barrier()


def _finalize_body(a_ref, b_ref, c_ref, d_ref, o_ref):
  for q in range(4):
    o_ref[:, Q * q:Q * (q + 1)] = 0.25 * (
        a_ref[q] + b_ref[q] + c_ref[q] + d_ref[q])


def _qgather_body(light_hbm, users_hbm, o_hbm, idxv, rowsv, sem):
  wid = lax.axis_index("s") * NC + lax.axis_index("c")
  bpw = 1024 // (NC * NS)
  base = wid * bpw
  pltpu.sync_copy(users_hbm.at[pl.ds(base, bpw)], idxv)
  pltpu.async_copy(light_hbm.at[idxv], rowsv, sem).wait()
  pltpu.sync_copy(rowsv, o_hbm.at[pl.ds(base, bpw)])


def _rating_body(q_ref, it_ref, o_ref):
  s = lax.dot_general(q_ref[...], it_ref[...], (((1,), (1,)), ((), ())),
                      precision=lax.Precision.HIGHEST)
  o_ref[...] = jax.nn.sigmoid(s)


def kernel(users, user_text, item_text, user_image, item_image, user_id_w,
           item_id_w, w1, b1, w2, b2, wi1, bi1, wi2, bi2,
           edge_row, edge_col, edge_w):
  f32 = jnp.float32
  text = jnp.concatenate([user_text, item_text], axis=0)
  image = jnp.concatenate([user_image, item_image], axis=0)
  idw = jnp.concatenate([user_id_w, item_id_w], axis=0)
  ht = w1.shape[1]
  htp = 256
  w1p = jnp.pad(w1, ((0, 0), (0, htp - ht)))
  b1p = jnp.pad(b1, (0, htp - ht)).reshape(1, htp)
  w2p = jnp.pad(w2, ((0, htp - ht), (0, 0)))
  b2p = b2.reshape(1, LAT)
  hi = wi1.shape[1]
  bi1p = bi1.reshape(1, hi)
  bi2p = bi2.reshape(1, LAT)

  x0 = pl.pallas_call(
      _embed_body,
      grid=(NN // RB,),
      in_specs=[
          pl.BlockSpec((RB, text.shape[1]), lambda i: (i, 0)),
          pl.BlockSpec((RB, image.shape[1]), lambda i: (i, 0)),
          pl.BlockSpec((RB, LAT), lambda i: (i, 0)),
          pl.BlockSpec((text.shape[1], htp), lambda i: (0, 0)),
          pl.BlockSpec((1, htp), lambda i: (0, 0)),
          pl.BlockSpec((htp, LAT), lambda i: (0, 0)),
          pl.BlockSpec((1, LAT), lambda i: (0, 0)),
          pl.BlockSpec((image.shape[1], hi), lambda i: (0, 0)),
          pl.BlockSpec((1, hi), lambda i: (0, 0)),
          pl.BlockSpec((hi, LAT), lambda i: (0, 0)),
          pl.BlockSpec((1, LAT), lambda i: (0, 0)),
      ],
      out_specs=pl.BlockSpec((4, RB, Q), lambda i: (0, i, 0)),
      out_shape=jax.ShapeDtypeStruct((4, NN, Q), f32),
  )(text, image, idw, w1p, b1p, w2p, b2p, wi1, bi1p, wi2, bi2p)

  mesh = plsc.VectorSubcoreMesh(core_axis_name="c", subcore_axis_name="s",
                                num_cores=NC, num_subcores=NS)
  spmm = functools.partial(
      pl.kernel,
      out_type=jax.ShapeDtypeStruct((4, NN, Q), f32),
      mesh=mesh,
      scratch_types=[
          pltpu.VMEM((EC,), jnp.int32),
          pltpu.VMEM((EC,), jnp.int32),
          pltpu.VMEM((EC,), f32),
          pltpu.VMEM((EC, Q), f32),
          pltpu.VMEM((ZR, Q), f32),
          pltpu.VMEM_SHARED((NN, Q), f32),
          pltpu.SemaphoreType.DMA,
      ],
  )(_spmm_body)

  x1 = spmm(x0, edge_row, edge_col, edge_w)
  x2 = spmm(x1, edge_row, edge_col, edge_w)
  x3 = spmm(x2, edge_row, edge_col, edge_w)

  light = pl.pallas_call(
      _finalize_body,
      grid=(NN // RB,),
      in_specs=[pl.BlockSpec((4, RB, Q), lambda i: (0, i, 0))] * 4,
      out_specs=pl.BlockSpec((RB, 4 * Q), lambda i: (i, 0)),
      out_shape=jax.ShapeDtypeStruct((NN, 4 * Q), f32),
  )(x0, x1, x2, x3)

  qu = pl.kernel(
      _qgather_body,
      out_type=jax.ShapeDtypeStruct((1024, 4 * Q), f32),
      mesh=mesh,
      scratch_types=[
          pltpu.VMEM((1024 // (NC * NS),), jnp.int32),
          pltpu.VMEM((1024 // (NC * NS), 4 * Q), f32),
          pltpu.SemaphoreType.DMA,
      ],
  )(light, users)

  items = light[NU:]
  IB = 512
  rating = pl.pallas_call(
      _rating_body,
      grid=(pl.cdiv(NI, IB),),
      in_specs=[
          pl.BlockSpec((1024, 4 * Q), lambda j: (0, 0)),
          pl.BlockSpec((IB, 4 * Q), lambda j: (j, 0)),
      ],
      out_specs=pl.BlockSpec((1024, IB), lambda j: (0, j)),
      out_shape=jax.ShapeDtypeStruct((1024, NI), f32),
  )(qu, items)

  return (rating, light[:NU], items)


# speedup vs baseline: 4.2950x; 4.2950x over previous
import functools

import jax
import jax.numpy as jnp
from jax import lax
from jax.experimental import pallas as pl
from jax.experimental.pallas import tpu as pltpu
from jax.experimental.pallas import tpu_sc as plsc

NU, NI = 20000, 30000
NN = NU + NI
NE = 800000
LAT = 64
Q = 16
NG = 8
RB = 1000
NC, NS = 2, 16
EPT = NE // NS
EC = 2000
ZR = 1000
NCH = NN // ZR
ZPS = 4


def _leaky(x):
  return jnp.where(x > 0, x, 0.01 * x)


def _l2n(x):
  n = jnp.sqrt(jnp.sum(x * x, axis=-1, keepdims=True))
  return x / jnp.maximum(n, 1e-12)


def _embed_body(t_ref, im_ref, id_ref, w1_ref, b1_ref, w2_ref, b2_ref,
                wi1_ref, bi1_ref, wi2_ref, bi2_ref, o_ref):
  hp = lax.Precision.HIGHEST
  h = lax.dot_general(t_ref[...], w1_ref[...], (((1,), (0,)), ((), ())),
                      precision=hp) + b1_ref[...]
  et = lax.dot_general(_leaky(h), w2_ref[...], (((1,), (0,)), ((), ())),
                       precision=hp) + b2_ref[...]
  hi = lax.dot_general(im_ref[...], wi1_ref[...], (((1,), (0,)), ((), ())),
                       precision=hp) + bi1_ref[...]
  ei = lax.dot_general(_leaky(hi), wi2_ref[...], (((1,), (0,)), ((), ())),
                       precision=hp) + bi2_ref[...]
  mm = _l2n(et) + _l2n(ei)
  for g in range(4):
    o_ref[g] = mm[:, Q * g:Q * (g + 1)]
  for g in range(4):
    o_ref[4 + g] = id_ref[:, Q * g:Q * (g + 1)]


def _spmm_body(x_hbm, er_hbm, ec_hbm, ew_hbm, y_hbm,
               colv, rowv, wv, rows, zbuf, acc, sem):
  c = lax.axis_index("c")
  s = lax.axis_index("s")
  ebase = s * EPT

  def zb(j, _):
    zbuf[j, pl.ds(0, Q)] = jnp.zeros((Q,), jnp.float32)
    return 0
  lax.fori_loop(0, ZR, zb, 0, unroll=8)

  for p in range(NG // 2):
    q = c + 2 * p
    for z in range(ZPS):
      cid = s + NS * z

      @pl.when(cid < NCH)
      def _():
        off = pl.multiple_of(cid * ZR, 8)
        pltpu.sync_copy(zbuf, acc.at[pl.ds(off, ZR)])
    plsc.subcore_barrier()

    def chunk(i, _):
      base = ebase + i * EC
      pltpu.sync_copy(ec_hbm.at[pl.ds(base, EC)], colv)
      pltpu.sync_copy(er_hbm.at[pl.ds(base, EC)], rowv)
      pltpu.sync_copy(ew_hbm.at[pl.ds(base, EC)], wv)
      pltpu.async_copy(x_hbm.at[q].at[colv], rows, sem).wait()

      def scale(j16, _):
        w16 = wv[pl.ds(j16 * 16, 16)]
        for k in range(16):
          ws = jnp.full((16,), w16[k], jnp.float32)
          j = j16 * 16 + k
          rows[j, pl.ds(0, Q)] = rows[j, pl.ds(0, Q)] * ws
        return 0
      lax.fori_loop(0, EC // 16, scale, 0)

      pltpu.sync_copy(rows, acc.at[rowv], add=True)
      return 0
    lax.fori_loop(0, EPT // EC, chunk, 0)
    plsc.subcore_barrier()

    for z in range(ZPS):
      cid = s + NS * z

      @pl.when(cid < NCH)
      def _():
        off = pl.multiple_of(cid * ZR, 8)
        pltpu.sync_copy(acc.at[pl.ds(off, ZR)], y_hbm.at[q].at[pl.ds(off, ZR)])
    plsc.subcore_barrier()


def _finalize_body(a_ref, b_ref, c_ref, d_ref, o_ref):
  for g in range(NG):
    o_ref[:, Q * g:Q * (g + 1)] = 0.25 * (
        a_ref[g] + b_ref[g] + c_ref[g] + d_ref[g])


def _qgather_body(light_hbm, users_hbm, o_hbm, idxv, rowsv, sem):
  wid = lax.axis_index("s") * NC + lax.axis_index("c")
  bpw = 1024 // (NC * NS)
  base = wid * bpw
  pltpu.sync_copy(users_hbm.at[pl.ds(base, bpw)], idxv)
  pltpu.async_copy(light_hbm.at[idxv], rowsv, sem).wait()
  pltpu.sync_copy(rowsv, o_hbm.at[pl.ds(base, bpw)])


def _rating_body(q_ref, it_ref, o_ref):
  s = lax.dot_general(q_ref[...], it_ref[...], (((1,), (1,)), ((), ())),
                      precision=lax.Precision.HIGHEST)
  o_ref[...] = jax.nn.sigmoid(s)


def kernel(users, user_text, item_text, user_image, item_image, user_id_w,
           item_id_w, w1, b1, w2, b2, wi1, bi1, wi2, bi2,
           edge_row, edge_col, edge_w):
  f32 = jnp.float32
  text = jnp.concatenate([user_text, item_text], axis=0)
  image = jnp.concatenate([user_image, item_image], axis=0)
  idw = jnp.concatenate([user_id_w, item_id_w], axis=0)
  ht = w1.shape[1]
  htp = 256
  w1p = jnp.pad(w1, ((0, 0), (0, htp - ht)))
  b1p = jnp.pad(b1, (0, htp - ht)).reshape(1, htp)
  w2p = jnp.pad(w2, ((0, htp - ht), (0, 0)))
  b2p = b2.reshape(1, LAT)
  hi = wi1.shape[1]
  bi1p = bi1.reshape(1, hi)
  bi2p = bi2.reshape(1, LAT)

  x0 = pl.pallas_call(
      _embed_body,
      grid=(NN // RB,),
      in_specs=[
          pl.BlockSpec((RB, text.shape[1]), lambda i: (i, 0)),
          pl.BlockSpec((RB, image.shape[1]), lambda i: (i, 0)),
          pl.BlockSpec((RB, LAT), lambda i: (i, 0)),
          pl.BlockSpec((text.shape[1], htp), lambda i: (0, 0)),
          pl.BlockSpec((1, htp), lambda i: (0, 0)),
          pl.BlockSpec((htp, LAT), lambda i: (0, 0)),
          pl.BlockSpec((1, LAT), lambda i: (0, 0)),
          pl.BlockSpec((image.shape[1], hi), lambda i: (0, 0)),
          pl.BlockSpec((1, hi), lambda i: (0, 0)),
          pl.BlockSpec((hi, LAT), lambda i: (0, 0)),
          pl.BlockSpec((1, LAT), lambda i: (0, 0)),
      ],
      out_specs=pl.BlockSpec((NG, RB, Q), lambda i: (0, i, 0)),
      out_shape=jax.ShapeDtypeStruct((NG, NN, Q), f32),
  )(text, image, idw, w1p, b1p, w2p, b2p, wi1, bi1p, wi2, bi2p)

  mesh = plsc.VectorSubcoreMesh(core_axis_name="c", subcore_axis_name="s",
                                num_cores=NC, num_subcores=NS)
  spmm = functools.partial(
      pl.kernel,
      out_type=jax.ShapeDtypeStruct((NG, NN, Q), f32),
      mesh=mesh,
      scratch_types=[
          pltpu.VMEM((EC,), jnp.int32),
          pltpu.VMEM((EC,), jnp.int32),
          pltpu.VMEM((EC,), f32),
          pltpu.VMEM((EC, Q), f32),
          pltpu.VMEM((ZR, Q), f32),
          pltpu.VMEM_SHARED((NN, Q), f32),
          pltpu.SemaphoreType.DMA,
      ],
      compiler_params=pltpu.CompilerParams(use_tc_tiling_on_sc=False),
  )(_spmm_body)

  x1 = spmm(x0, edge_row, edge_col, edge_w)
  x2 = spmm(x1, edge_row, edge_col, edge_w)
  x3 = spmm(x2, edge_row, edge_col, edge_w)

  light = pl.pallas_call(
      _finalize_body,
      grid=(NN // RB,),
      in_specs=[pl.BlockSpec((NG, RB, Q), lambda i: (0, i, 0))] * 4,
      out_specs=pl.BlockSpec((RB, NG * Q), lambda i: (i, 0)),
      out_shape=jax.ShapeDtypeStruct((NN, NG * Q), f32),
  )(x0, x1, x2, x3)

  qu = pl.kernel(
      _qgather_body,
      out_type=jax.ShapeDtypeStruct((1024, NG * Q), f32),
      mesh=mesh,
      scratch_types=[
          pltpu.VMEM((1024 // (NC * NS),), jnp.int32),
          pltpu.VMEM((1024 // (NC * NS), NG * Q), f32),
          pltpu.SemaphoreType.DMA,
      ],
  )(light, users)

  items = light[NU:]
  IB = 512
  rating = pl.pallas_call(
      _rating_body,
      grid=(pl.cdiv(NI, IB),),
      in_specs=[
          pl.BlockSpec((1024, NG * Q), lambda j: (0, 0)),
          pl.BlockSpec((IB, NG * Q), lambda j: (j, 0)),
      ],
      out_specs=pl.BlockSpec((1024, IB), lambda j: (0, j)),
      out_shape=jax.ShapeDtypeStruct((1024, NI), f32),
  )(qu, items)

  return (rating, light[:NU], items)


# speedup vs baseline: 4.3502x; 1.0129x over previous
import functools

import jax
import jax.numpy as jnp
from jax import lax
from jax.experimental import pallas as pl
from jax.experimental.pallas import tpu as pltpu
from jax.experimental.pallas import tpu_sc as plsc

NU, NI = 20000, 30000
NN = NU + NI
NE = 800000
LAT = 64
Q = 16
NG = 8
RB = 1000
NC, NS = 2, 16
EPT = NE // NS
EC = 2000
ZR = 1000
NCH = NN // ZR
ZPS = 4


def _leaky(x):
  return jnp.where(x > 0, x, 0.01 * x)


def _l2n(x):
  n = jnp.sqrt(jnp.sum(x * x, axis=-1, keepdims=True))
  return x / jnp.maximum(n, 1e-12)


def _embed_body(t_ref, im_ref, id_ref, w1_ref, b1_ref, w2_ref, b2_ref,
                wi1_ref, bi1_ref, wi2_ref, bi2_ref, o_ref):
  hp = lax.Precision.HIGHEST
  h = lax.dot_general(t_ref[...], w1_ref[...], (((1,), (0,)), ((), ())),
                      precision=hp) + b1_ref[...]
  et = lax.dot_general(_leaky(h), w2_ref[...], (((1,), (0,)), ((), ())),
                       precision=hp) + b2_ref[...]
  hi = lax.dot_general(im_ref[...], wi1_ref[...], (((1,), (0,)), ((), ())),
                       precision=hp) + bi1_ref[...]
  ei = lax.dot_general(_leaky(hi), wi2_ref[...], (((1,), (0,)), ((), ())),
                       precision=hp) + bi2_ref[...]
  mm = _l2n(et) + _l2n(ei)
  for g in range(4):
    o_ref[g] = mm[:, Q * g:Q * (g + 1)]
  for g in range(4):
    o_ref[4 + g] = id_ref[:, Q * g:Q * (g + 1)]


def _spmm_body(x_hbm, er_hbm, ec_hbm, ew_hbm, y_hbm,
               colv, rowv, wv, rows, zbuf, acc, sem):
  c = lax.axis_index("c")
  s = lax.axis_index("s")
  ebase = s * EPT

  def zb(j, _):
    zbuf[j, pl.ds(0, Q)] = jnp.zeros((Q,), jnp.float32)
    return 0
  lax.fori_loop(0, ZR, zb, 0, unroll=8)

  for p in range(NG // 2):
    q = c + 2 * p
    for z in range(ZPS):
      cid = s + NS * z

      @pl.when(cid < NCH)
      def _():
        off = pl.multiple_of(cid * ZR, 8)
        pltpu.sync_copy(zbuf, acc.at[pl.ds(off, ZR)])
    plsc.subcore_barrier()

    def chunk(i, _):
      base = ebase + i * EC
      pltpu.sync_copy(ec_hbm.at[pl.ds(base, EC)], colv)
      pltpu.sync_copy(er_hbm.at[pl.ds(base, EC)], rowv)
      pltpu.sync_copy(ew_hbm.at[pl.ds(base, EC)], wv)
      pltpu.async_copy(x_hbm.at[q].at[colv], rows, sem).wait()

      def scale(j16, _):
        w16 = wv[pl.ds(j16 * 16, 16)]
        for k in range(16):
          ws = jnp.full((16,), w16[k], jnp.float32)
          jj = j16 * 16 + k
          rows[jj, pl.ds(0, Q)] = rows[jj, pl.ds(0, Q)] * ws
        return 0
      lax.fori_loop(0, EC // 16, scale, 0)

      pltpu.sync_copy(rows, acc.at[rowv], add=True)
      return 0
    lax.fori_loop(0, EPT // EC, chunk, 0)
    plsc.subcore_barrier()

    for z in range(ZPS):
      cid = s + NS * z

      @pl.when(cid < NCH)
      def _():
        off = pl.multiple_of(cid * ZR, 8)
        pltpu.sync_copy(acc.at[pl.ds(off, ZR)], y_hbm.at[q].at[pl.ds(off, ZR)])
    plsc.subcore_barrier()


def _finalize_body(a_ref, b_ref, c_ref, d_ref, o_ref):
  for g in range(NG):
    o_ref[:, Q * g:Q * (g + 1)] = 0.25 * (
        a_ref[g] + b_ref[g] + c_ref[g] + d_ref[g])


def _qgather_body(light_hbm, users_hbm, o_hbm, idxv, rowsv, sem):
  wid = lax.axis_index("s") * NC + lax.axis_index("c")
  bpw = 1024 // (NC * NS)
  base = wid * bpw
  pltpu.sync_copy(users_hbm.at[pl.ds(base, bpw)], idxv)
  pltpu.async_copy(light_hbm.at[idxv], rowsv, sem).wait()
  pltpu.sync_copy(rowsv, o_hbm.at[pl.ds(base, bpw)])


def _rating_body(q_ref, it_ref, o_ref):
  s = lax.dot_general(q_ref[...], it_ref[...], (((1,), (1,)), ((), ())),
                      precision=lax.Precision.HIGHEST)
  o_ref[...] = jax.nn.sigmoid(s)


def kernel(users, user_text, item_text, user_image, item_image, user_id_w,
           item_id_w, w1, b1, w2, b2, wi1, bi1, wi2, bi2,
           edge_row, edge_col, edge_w):
  f32 = jnp.float32
  ht = w1.shape[1]
  htp = 256
  w1p = jnp.pad(w1, ((0, 0), (0, htp - ht)))
  b1p = jnp.pad(b1, (0, htp - ht)).reshape(1, htp)
  w2p = jnp.pad(w2, ((0, htp - ht), (0, 0)))
  b2p = b2.reshape(1, LAT)
  hi = wi1.shape[1]
  bi1p = bi1.reshape(1, hi)
  bi2p = bi2.reshape(1, LAT)

  def embed(textA, imgA, idA, n):
    return pl.pallas_call(
        _embed_body,
        grid=(n // RB,),
        in_specs=[
            pl.BlockSpec((RB, textA.shape[1]), lambda i: (i, 0)),
            pl.BlockSpec((RB, imgA.shape[1]), lambda i: (i, 0)),
            pl.BlockSpec((RB, LAT), lambda i: (i, 0)),
            pl.BlockSpec((textA.shape[1], htp), lambda i: (0, 0)),
            pl.BlockSpec((1, htp), lambda i: (0, 0)),
            pl.BlockSpec((htp, LAT), lambda i: (0, 0)),
            pl.BlockSpec((1, LAT), lambda i: (0, 0)),
            pl.BlockSpec((imgA.shape[1], hi), lambda i: (0, 0)),
            pl.BlockSpec((1, hi), lambda i: (0, 0)),
            pl.BlockSpec((hi, LAT), lambda i: (0, 0)),
            pl.BlockSpec((1, LAT), lambda i: (0, 0)),
        ],
        out_specs=pl.BlockSpec((NG, RB, Q), lambda i: (0, i, 0)),
        out_shape=jax.ShapeDtypeStruct((NG, n, Q), f32),
    )(textA, imgA, idA, w1p, b1p, w2p, b2p, wi1, bi1p, wi2, bi2p)

  xu = embed(user_text, user_image, user_id_w, NU)
  xi = embed(item_text, item_image, item_id_w, NI)
  x0 = jnp.concatenate([xu, xi], axis=1)

  mesh = plsc.VectorSubcoreMesh(core_axis_name="c", subcore_axis_name="s",
                                num_cores=NC, num_subcores=NS)
  spmm = functools.partial(
      pl.kernel,
      out_type=jax.ShapeDtypeStruct((NG, NN, Q), f32),
      mesh=mesh,
      scratch_types=[
          pltpu.VMEM((EC,), jnp.int32),
          pltpu.VMEM((EC,), jnp.int32),
          pltpu.VMEM((EC,), f32),
          pltpu.VMEM((EC, Q), f32),
          pltpu.VMEM((ZR, Q), f32),
          pltpu.VMEM_SHARED((NN, Q), f32),
          pltpu.SemaphoreType.DMA,
      ],
      compiler_params=pltpu.CompilerParams(use_tc_tiling_on_sc=False),
  )(_spmm_body)

  x1 = spmm(x0, edge_row, edge_col, edge_w)
  x2 = spmm(x1, edge_row, edge_col, edge_w)
  x3 = spmm(x2, edge_row, edge_col, edge_w)

  def finalize(n, off):
    return pl.pallas_call(
        _finalize_body,
        grid=(n // RB,),
        in_specs=[pl.BlockSpec((NG, RB, Q), lambda i: (0, i + off, 0))] * 4,
        out_specs=pl.BlockSpec((RB, NG * Q), lambda i: (i, 0)),
        out_shape=jax.ShapeDtypeStruct((n, NG * Q), f32),
    )(x0, x1, x2, x3)

  all_users = finalize(NU, 0)
  all_items = finalize(NI, NU // RB)

  qu = pl.kernel(
      _qgather_body,
      out_type=jax.ShapeDtypeStruct((1024, NG * Q), f32),
      mesh=mesh,
      scratch_types=[
          pltpu.VMEM((1024 // (NC * NS),), jnp.int32),
          pltpu.VMEM((1024 // (NC * NS), NG * Q), f32),
          pltpu.SemaphoreType.DMA,
      ],
  )(all_users, users)

  IB = 512
  rating = pl.pallas_call(
      _rating_body,
      grid=(pl.cdiv(NI, IB),),
      in_specs=[
          pl.BlockSpec((1024, NG * Q), lambda j: (0, 0)),
          pl.BlockSpec((IB, NG * Q), lambda j: (j, 0)),
      ],
      out_specs=pl.BlockSpec((1024, IB), lambda j: (0, j)),
      out_shape=jax.ShapeDtypeStruct((1024, NI), f32),
  )(qu, all_items)

  return (rating, all_users, all_items)


# speedup vs baseline: 4.5584x; 1.0479x over previous
import functools

import jax
import jax.numpy as jnp
from jax import lax
from jax.experimental import pallas as pl
from jax.experimental.pallas import tpu as pltpu
from jax.experimental.pallas import tpu_sc as plsc

NU, NI = 20000, 30000
NN = NU + NI
NE = 800000
LAT = 64
Q = 16
NG = 8
RB = 1000
NC, NS = 2, 16
EPT = NE // NS
EC = 2000
ZR = 1000
NCH = NN // ZR
ZPS = 4


def _leaky(x):
  return jnp.where(x > 0, x, 0.01 * x)


def _l2n(x):
  n = jnp.sqrt(jnp.sum(x * x, axis=-1, keepdims=True))
  return x / jnp.maximum(n, 1e-12)


def _embed_body(t_ref, im_ref, id_ref, w1_ref, b1_ref, w2_ref, b2_ref,
                wi1_ref, bi1_ref, wi2_ref, bi2_ref, o_ref):
  hp = lax.Precision.HIGHEST
  h = lax.dot_general(t_ref[...], w1_ref[...], (((1,), (0,)), ((), ())),
                      precision=hp) + b1_ref[...]
  et = lax.dot_general(_leaky(h), w2_ref[...], (((1,), (0,)), ((), ())),
                       precision=hp) + b2_ref[...]
  hi = lax.dot_general(im_ref[...], wi1_ref[...], (((1,), (0,)), ((), ())),
                       precision=hp) + bi1_ref[...]
  ei = lax.dot_general(_leaky(hi), wi2_ref[...], (((1,), (0,)), ((), ())),
                       precision=hp) + bi2_ref[...]
  mm = _l2n(et) + _l2n(ei)
  for g in range(4):
    o_ref[g] = mm[:, Q * g:Q * (g + 1)]
  for g in range(4):
    o_ref[4 + g] = id_ref[:, Q * g:Q * (g + 1)]


def _spmm3_body(x_hbm, er_hbm, ec_hbm, ew_hbm, y1_hbm, y2_hbm, y3_hbm,
                colv0, rowv0, wv0, rows0, gsem0, ssem0,
                colv1, rowv1, wv1, rows1, gsem1, ssem1,
                zbuf, acc):
  c = lax.axis_index("c")
  s = lax.axis_index("s")
  ebase = s * EPT
  bufs = ((colv0, rowv0, wv0, rows0, gsem0, ssem0),
          (colv1, rowv1, wv1, rows1, gsem1, ssem1))

  def zb(j, _):
    zbuf[j, pl.ds(0, Q)] = jnp.zeros((Q,), jnp.float32)
    return 0
  lax.fori_loop(0, ZR, zb, 0, unroll=8)

  def start_half(i2, q, src, b):
    colv, rowv, wv, rows, gsem, _ = bufs[b]
    base = ebase + i2 * EC
    pltpu.sync_copy(ec_hbm.at[pl.ds(base, EC)], colv)
    gd = pltpu.async_copy(src.at[q].at[colv], rows, gsem)
    pltpu.sync_copy(er_hbm.at[pl.ds(base, EC)], rowv)
    pltpu.sync_copy(ew_hbm.at[pl.ds(base, EC)], wv)
    return gd

  def work_half(gd, b):
    _, rowv, wv, rows, _, ssem = bufs[b]
    gd.wait()

    def scale(j16, _):
      w16 = wv[pl.ds(j16 * 16, 16)]
      for k in range(16):
        ws = jnp.full((16,), w16[k], jnp.float32)
        jj = j16 * 16 + k
        rows[jj, pl.ds(0, Q)] = rows[jj, pl.ds(0, Q)] * ws
      return 0
    lax.fori_loop(0, EC // 16, scale, 0)
    return pltpu.async_copy(rows, acc.at[rowv], ssem, add=True)

  for p in range(NG // 2):
    q = c + 2 * p
    for src, dst in ((x_hbm, y1_hbm), (y1_hbm, y2_hbm), (y2_hbm, y3_hbm)):
      for z in range(ZPS):
        cid = s + NS * z

        @pl.when(cid < NCH)
        def _():
          off = pl.multiple_of(cid * ZR, 8)
          pltpu.sync_copy(zbuf, acc.at[pl.ds(off, ZR)])
      plsc.subcore_barrier()

      def chunk1(i1, _):
        g0 = start_half(i1, q, src, 0)
        s0 = work_half(g0, 0)
        s0.wait()
        return 0
      lax.fori_loop(0, EPT // EC, chunk1, 0)
      plsc.subcore_barrier()

      for z in range(ZPS):
        cid = s + NS * z

        @pl.when(cid < NCH)
        def _():
          off = pl.multiple_of(cid * ZR, 8)
          pltpu.sync_copy(acc.at[pl.ds(off, ZR)], dst.at[q].at[pl.ds(off, ZR)])
      plsc.subcore_barrier()


def _finalize_body(a_ref, b_ref, c_ref, d_ref, o_ref):
  for g in range(NG):
    o_ref[:, Q * g:Q * (g + 1)] = 0.25 * (
        a_ref[g] + b_ref[g] + c_ref[g] + d_ref[g])


def _qgather_body(light_hbm, users_hbm, o_hbm, idxv, rowsv, sem):
  wid = lax.axis_index("s") * NC + lax.axis_index("c")
  bpw = 1024 // (NC * NS)
  base = wid * bpw
  pltpu.sync_copy(users_hbm.at[pl.ds(base, bpw)], idxv)
  pltpu.async_copy(light_hbm.at[idxv], rowsv, sem).wait()
  pltpu.sync_copy(rowsv, o_hbm.at[pl.ds(base, bpw)])


def _rating_body(q_ref, it_ref, o_ref):
  s = lax.dot_general(q_ref[...], it_ref[...], (((1,), (1,)), ((), ())),
                      precision=lax.Precision.HIGHEST)
  o_ref[...] = jax.nn.sigmoid(s)


def kernel(users, user_text, item_text, user_image, item_image, user_id_w,
           item_id_w, w1, b1, w2, b2, wi1, bi1, wi2, bi2,
           edge_row, edge_col, edge_w):
  f32 = jnp.float32
  ht = w1.shape[1]
  htp = 256
  w1p = jnp.pad(w1, ((0, 0), (0, htp - ht)))
  b1p = jnp.pad(b1, (0, htp - ht)).reshape(1, htp)
  w2p = jnp.pad(w2, ((0, htp - ht), (0, 0)))
  b2p = b2.reshape(1, LAT)
  hi = wi1.shape[1]
  bi1p = bi1.reshape(1, hi)
  bi2p = bi2.reshape(1, LAT)

  def embed(textA, imgA, idA, n):
    return pl.pallas_call(
        _embed_body,
        grid=(n // RB,),
        in_specs=[
            pl.BlockSpec((RB, textA.shape[1]), lambda i: (i, 0)),
            pl.BlockSpec((RB, imgA.shape[1]), lambda i: (i, 0)),
            pl.BlockSpec((RB, LAT), lambda i: (i, 0)),
            pl.BlockSpec((textA.shape[1], htp), lambda i: (0, 0)),
            pl.BlockSpec((1, htp), lambda i: (0, 0)),
            pl.BlockSpec((htp, LAT), lambda i: (0, 0)),
            pl.BlockSpec((1, LAT), lambda i: (0, 0)),
            pl.BlockSpec((imgA.shape[1], hi), lambda i: (0, 0)),
            pl.BlockSpec((1, hi), lambda i: (0, 0)),
            pl.BlockSpec((hi, LAT), lambda i: (0, 0)),
            pl.BlockSpec((1, LAT), lambda i: (0, 0)),
        ],
        out_specs=pl.BlockSpec((NG, RB, Q), lambda i: (0, i, 0)),
        out_shape=jax.ShapeDtypeStruct((NG, n, Q), f32),
    )(textA, imgA, idA, w1p, b1p, w2p, b2p, wi1, bi1p, wi2, bi2p)

  xu = embed(user_text, user_image, user_id_w, NU)
  xi = embed(item_text, item_image, item_id_w, NI)
  x0 = jnp.concatenate([xu, xi], axis=1)

  mesh = plsc.VectorSubcoreMesh(core_axis_name="c", subcore_axis_name="s",
                                num_cores=NC, num_subcores=NS)
  spmm3 = functools.partial(
      pl.kernel,
      out_type=(jax.ShapeDtypeStruct((NG, NN, Q), f32),) * 3,
      mesh=mesh,
      scratch_types=[
          pltpu.VMEM((EC,), jnp.int32),
          pltpu.VMEM((EC,), jnp.int32),
          pltpu.VMEM((EC,), f32),
          pltpu.VMEM((EC, Q), f32),
          pltpu.SemaphoreType.DMA,
          pltpu.SemaphoreType.DMA,
          pltpu.VMEM((EC,), jnp.int32),
          pltpu.VMEM((EC,), jnp.int32),
          pltpu.VMEM((EC,), f32),
          pltpu.VMEM((EC, Q), f32),
          pltpu.SemaphoreType.DMA,
          pltpu.SemaphoreType.DMA,
          pltpu.VMEM((ZR, Q), f32),
          pltpu.VMEM_SHARED((NN, Q), f32),
      ],
      compiler_params=pltpu.CompilerParams(use_tc_tiling_on_sc=False),
  )(_spmm3_body)

  x1, x2, x3 = spmm3(x0, edge_row, edge_col, edge_w)

  def finalize(n, off):
    return pl.pallas_call(
        _finalize_body,
        grid=(n // RB,),
        in_specs=[pl.BlockSpec((NG, RB, Q), lambda i: (0, i + off, 0))] * 4,
        out_specs=pl.BlockSpec((RB, NG * Q), lambda i: (i, 0)),
        out_shape=jax.ShapeDtypeStruct((n, NG * Q), f32),
    )(x0, x1, x2, x3)

  all_users = finalize(NU, 0)
  all_items = finalize(NI, NU // RB)

  qu = pl.kernel(
      _qgather_body,
      out_type=jax.ShapeDtypeStruct((1024, NG * Q), f32),
      mesh=mesh,
      scratch_types=[
          pltpu.VMEM((1024 // (NC * NS),), jnp.int32),
          pltpu.VMEM((1024 // (NC * NS), NG * Q), f32),
          pltpu.SemaphoreType.DMA,
      ],
  )(all_users, users)

  IB = 512
  rating = pl.pallas_call(
      _rating_body,
      grid=(pl.cdiv(NI, IB),),
      in_specs=[
          pl.BlockSpec((1024, NG * Q), lambda j: (0, 0)),
          pl.BlockSpec((IB, NG * Q), lambda j: (j, 0)),
      ],
      out_specs=pl.BlockSpec((1024, IB), lambda j: (0, j)),
      out_shape=jax.ShapeDtypeStruct((1024, NI), f32),
  )(qu, all_items)

  return (rating, all_users, all_items)


# speedup vs baseline: 4.9717x; 1.0907x over previous
import functools

import jax
import jax.numpy as jnp
from jax import lax
from jax.experimental import pallas as pl
from jax.experimental.pallas import tpu as pltpu
from jax.experimental.pallas import tpu_sc as plsc

NU, NI = 20000, 30000
NN = NU + NI
NE = 800000
LAT = 64
Q = 16
NG = 8
RB = 1000
NC, NS = 2, 16
EPT = NE // NS
EC = 1248
ECT = 80
NCK = 40
ZR = 1000
NCH = NN // ZR
ZPS = 4


def _leaky(x):
  return jnp.where(x > 0, x, 0.01 * x)


def _l2n(x):
  n = jnp.sqrt(jnp.sum(x * x, axis=-1, keepdims=True))
  return x / jnp.maximum(n, 1e-12)


def _embed_body(t_ref, im_ref, id_ref, w1_ref, b1_ref, w2_ref, b2_ref,
                wi1_ref, bi1_ref, wi2_ref, bi2_ref, o_ref):
  hp = lax.Precision.HIGHEST
  h = lax.dot_general(t_ref[...], w1_ref[...], (((1,), (0,)), ((), ())),
                      precision=hp) + b1_ref[...]
  et = lax.dot_general(_leaky(h), w2_ref[...], (((1,), (0,)), ((), ())),
                       precision=hp) + b2_ref[...]
  hi = lax.dot_general(im_ref[...], wi1_ref[...], (((1,), (0,)), ((), ())),
                       precision=hp) + bi1_ref[...]
  ei = lax.dot_general(_leaky(hi), wi2_ref[...], (((1,), (0,)), ((), ())),
                       precision=hp) + bi2_ref[...]
  mm = _l2n(et) + _l2n(ei)
  for g in range(4):
    o_ref[g] = mm[:, Q * g:Q * (g + 1)]
  for g in range(4):
    o_ref[4 + g] = id_ref[:, Q * g:Q * (g + 1)]


def _spmm3_body(x_hbm, er_hbm, ec_hbm, ew_hbm, y1_hbm, y2_hbm, y3_hbm,
                colv0, rowv0, wv0, rows0, gsem0, ssem0,
                colv1, rowv1, wv1, rows1, gsem1, ssem1,
                colvt, rowvt, wvt, rowst,
                zbuf, acc):
  c = lax.axis_index("c")
  s = lax.axis_index("s")
  ebase = s * EPT
  bufs = ((colv0, rowv0, wv0, rows0, gsem0, ssem0),
          (colv1, rowv1, wv1, rows1, gsem1, ssem1))

  def zb(j, _):
    zbuf[j, pl.ds(0, Q)] = jnp.zeros((Q,), jnp.float32)
    return 0
  lax.fori_loop(0, ZR, zb, 0, unroll=8)

  def start_chunk(i, q, src, b):
    colv, rowv, wv, rows, gsem, _ = bufs[b]
    base = ebase + i * EC
    pltpu.sync_copy(ec_hbm.at[pl.ds(base, EC)], colv)
    gd = pltpu.async_copy(src.at[q].at[colv], rows, gsem)
    pltpu.sync_copy(er_hbm.at[pl.ds(base, EC)], rowv)
    pltpu.sync_copy(ew_hbm.at[pl.ds(base, EC)], wv)
    return gd

  def work_chunk(gd, b):
    _, rowv, wv, rows, _, ssem = bufs[b]
    gd.wait()

    def scale(j16, _):
      w16 = wv[pl.ds(j16 * 16, 16)]
      for k in range(16):
        ws = jnp.full((16,), w16[k], jnp.float32)
        jj = j16 * 16 + k
        rows[jj, pl.ds(0, Q)] = rows[jj, pl.ds(0, Q)] * ws
      return 0
    lax.fori_loop(0, EC // 16, scale, 0)
    return pltpu.async_copy(rows, acc.at[rowv], ssem, add=True)

  def tail_chunk(q, src):
    base = ebase + NCK * EC
    pltpu.sync_copy(ec_hbm.at[pl.ds(base, ECT)], colvt)
    pltpu.sync_copy(er_hbm.at[pl.ds(base, ECT)], rowvt)
    pltpu.sync_copy(ew_hbm.at[pl.ds(base, ECT)], wvt)
    pltpu.async_copy(src.at[q].at[colvt], rowst, gsem0).wait()

    def scale(j16, _):
      w16 = wvt[pl.ds(j16 * 16, 16)]
      for k in range(16):
        ws = jnp.full((16,), w16[k], jnp.float32)
        jj = j16 * 16 + k
        rowst[jj, pl.ds(0, Q)] = rowst[jj, pl.ds(0, Q)] * ws
      return 0
    lax.fori_loop(0, ECT // 16, scale, 0)
    pltpu.sync_copy(rowst, acc.at[rowvt], add=True)

  for p in range(NG // 2):
    q = c + 2 * p
    for src, dst in ((x_hbm, y1_hbm), (y1_hbm, y2_hbm), (y2_hbm, y3_hbm)):
      for z in range(ZPS):
        cid = s + NS * z

        @pl.when(cid < NCH)
        def _():
          off = pl.multiple_of(cid * ZR, 8)
          pltpu.sync_copy(zbuf, acc.at[pl.ds(off, ZR)])
      plsc.subcore_barrier()

      def chunk2(i2, _):
        g0 = start_chunk(2 * i2, q, src, 0)
        g1 = start_chunk(2 * i2 + 1, q, src, 1)
        s0 = work_chunk(g0, 0)
        s1 = work_chunk(g1, 1)
        s0.wait()
        s1.wait()
        return 0
      lax.fori_loop(0, NCK // 2, chunk2, 0)
      tail_chunk(q, src)

      plsc.subcore_barrier()

      for z in range(ZPS):
        cid = s + NS * z

        @pl.when(cid < NCH)
        def _():
          off = pl.multiple_of(cid * ZR, 8)
          pltpu.sync_copy(acc.at[pl.ds(off, ZR)], dst.at[q].at[pl.ds(off, ZR)])
      plsc.subcore_barrier()


def _finalize_body(a_ref, b_ref, c_ref, d_ref, o_ref):
  for g in range(NG):
    o_ref[:, Q * g:Q * (g + 1)] = 0.25 * (
        a_ref[g] + b_ref[g] + c_ref[g] + d_ref[g])


def _qgather_body(light_hbm, users_hbm, o_hbm, idxv, rowsv, sem):
  wid = lax.axis_index("s") * NC + lax.axis_index("c")
  bpw = 1024 // (NC * NS)
  base = wid * bpw
  pltpu.sync_copy(users_hbm.at[pl.ds(base, bpw)], idxv)
  pltpu.async_copy(light_hbm.at[idxv], rowsv, sem).wait()
  pltpu.sync_copy(rowsv, o_hbm.at[pl.ds(base, bpw)])


def _rating_body(q_ref, it_ref, o_ref):
  s = lax.dot_general(q_ref[...], it_ref[...], (((1,), (1,)), ((), ())),
                      precision=lax.Precision.HIGHEST)
  o_ref[...] = jax.nn.sigmoid(s)


def kernel(users, user_text, item_text, user_image, item_image, user_id_w,
           item_id_w, w1, b1, w2, b2, wi1, bi1, wi2, bi2,
           edge_row, edge_col, edge_w):
  f32 = jnp.float32
  ht = w1.shape[1]
  htp = 256
  w1p = jnp.pad(w1, ((0, 0), (0, htp - ht)))
  b1p = jnp.pad(b1, (0, htp - ht)).reshape(1, htp)
  w2p = jnp.pad(w2, ((0, htp - ht), (0, 0)))
  b2p = b2.reshape(1, LAT)
  hi = wi1.shape[1]
  bi1p = bi1.reshape(1, hi)
  bi2p = bi2.reshape(1, LAT)

  def embed(textA, imgA, idA, n):
    return pl.pallas_call(
        _embed_body,
        grid=(n // RB,),
        in_specs=[
            pl.BlockSpec((RB, textA.shape[1]), lambda i: (i, 0)),
            pl.BlockSpec((RB, imgA.shape[1]), lambda i: (i, 0)),
            pl.BlockSpec((RB, LAT), lambda i: (i, 0)),
            pl.BlockSpec((textA.shape[1], htp), lambda i: (0, 0)),
            pl.BlockSpec((1, htp), lambda i: (0, 0)),
            pl.BlockSpec((htp, LAT), lambda i: (0, 0)),
            pl.BlockSpec((1, LAT), lambda i: (0, 0)),
            pl.BlockSpec((imgA.shape[1], hi), lambda i: (0, 0)),
            pl.BlockSpec((1, hi), lambda i: (0, 0)),
            pl.BlockSpec((hi, LAT), lambda i: (0, 0)),
            pl.BlockSpec((1, LAT), lambda i: (0, 0)),
        ],
        out_specs=pl.BlockSpec((NG, RB, Q), lambda i: (0, i, 0)),
        out_shape=jax.ShapeDtypeStruct((NG, n, Q), f32),
    )(textA, imgA, idA, w1p, b1p, w2p, b2p, wi1, bi1p, wi2, bi2p)

  xu = embed(user_text, user_image, user_id_w, NU)
  xi = embed(item_text, item_image, item_id_w, NI)
  x0 = jnp.concatenate([xu, xi], axis=1)

  mesh = plsc.VectorSubcoreMesh(core_axis_name="c", subcore_axis_name="s",
                                num_cores=NC, num_subcores=NS)
  spmm3 = functools.partial(
      pl.kernel,
      out_type=(jax.ShapeDtypeStruct((NG, NN, Q), f32),) * 3,
      mesh=mesh,
      scratch_types=[
          pltpu.VMEM((EC,), jnp.int32),
          pltpu.VMEM((EC,), jnp.int32),
          pltpu.VMEM((EC,), f32),
          pltpu.VMEM((EC, Q), f32),
          pltpu.SemaphoreType.DMA,
          pltpu.SemaphoreType.DMA,
          pltpu.VMEM((EC,), jnp.int32),
          pltpu.VMEM((EC,), jnp.int32),
          pltpu.VMEM((EC,), f32),
          pltpu.VMEM((EC, Q), f32),
          pltpu.SemaphoreType.DMA,
          pltpu.SemaphoreType.DMA,
          pltpu.VMEM((ECT,), jnp.int32),
          pltpu.VMEM((ECT,), jnp.int32),
          pltpu.VMEM((ECT,), f32),
          pltpu.VMEM((ECT, Q), f32),
          pltpu.VMEM((ZR, Q), f32),
          pltpu.VMEM_SHARED((NN, Q), f32),
      ],
      compiler_params=pltpu.CompilerParams(use_tc_tiling_on_sc=False),
  )(_spmm3_body)

  x1, x2, x3 = spmm3(x0, edge_row, edge_col, edge_w)

  def finalize(n, off):
    return pl.pallas_call(
        _finalize_body,
        grid=(n // RB,),
        in_specs=[pl.BlockSpec((NG, RB, Q), lambda i: (0, i + off, 0))] * 4,
        out_specs=pl.BlockSpec((RB, NG * Q), lambda i: (i, 0)),
        out_shape=jax.ShapeDtypeStruct((n, NG * Q), f32),
    )(x0, x1, x2, x3)

  all_users = finalize(NU, 0)
  all_items = finalize(NI, NU // RB)

  qu = pl.kernel(
      _qgather_body,
      out_type=jax.ShapeDtypeStruct((1024, NG * Q), f32),
      mesh=mesh,
      scratch_types=[
          pltpu.VMEM((1024 // (NC * NS),), jnp.int32),
          pltpu.VMEM((1024 // (NC * NS), NG * Q), f32),
          pltpu.SemaphoreType.DMA,
      ],
  )(all_users, users)

  IB = 512
  rating = pl.pallas_call(
      _rating_body,
      grid=(pl.cdiv(NI, IB),),
      in_specs=[
          pl.BlockSpec((1024, NG * Q), lambda j: (0, 0)),
          pl.BlockSpec((IB, NG * Q), lambda j: (j, 0)),
      ],
      out_specs=pl.BlockSpec((1024, IB), lambda j: (0, j)),
      out_shape=jax.ShapeDtypeStruct((1024, NI), f32),
  )(qu, all_items)

  return (rating, all_users, all_items)


# speedup vs baseline: 5.7321x; 1.1529x over previous
import functools

import jax
import jax.numpy as jnp
from jax import lax
from jax.experimental import pallas as pl
from jax.experimental.pallas import tpu as pltpu
from jax.experimental.pallas import tpu_sc as plsc

NU, NI = 20000, 30000
NN = NU + NI
NE = 800000
LAT = 64
Q = 16
NG = 8
RB = 1000
NC, NS = 2, 16
EPT = NE // NS
EC = 1248
ECT = 80
NCK = 40
ZR = 1000
NCH = NN // ZR
ZPS = 4


def _leaky(x):
  return jnp.where(x > 0, x, 0.01 * x)


def _l2n(x):
  n = jnp.sqrt(jnp.sum(x * x, axis=-1, keepdims=True))
  return x / jnp.maximum(n, 1e-12)


def _embed_body(t_ref, im_ref, id_ref, w1_ref, b1_ref, w2_ref, b2_ref,
                wi1_ref, bi1_ref, wi2_ref, bi2_ref, o_ref):
  hp = lax.Precision.DEFAULT
  h = lax.dot_general(t_ref[...], w1_ref[...], (((1,), (0,)), ((), ())),
                      precision=hp) + b1_ref[...]
  et = lax.dot_general(_leaky(h), w2_ref[...], (((1,), (0,)), ((), ())),
                       precision=hp) + b2_ref[...]
  hi = lax.dot_general(im_ref[...], wi1_ref[...], (((1,), (0,)), ((), ())),
                       precision=hp) + bi1_ref[...]
  ei = lax.dot_general(_leaky(hi), wi2_ref[...], (((1,), (0,)), ((), ())),
                       precision=hp) + bi2_ref[...]
  mm = _l2n(et) + _l2n(ei)
  for g in range(4):
    o_ref[g] = mm[:, Q * g:Q * (g + 1)]
  for g in range(4):
    o_ref[4 + g] = id_ref[:, Q * g:Q * (g + 1)]


def _spmm3_body(x_hbm, er_hbm, ec_hbm, ew_hbm, y1_hbm, y2_hbm, y3_hbm,
                colv0, rowv0, wv0, rows0, gsem0, ssem0,
                colv1, rowv1, wv1, rows1, gsem1, ssem1,
                colvt, rowvt, wvt, rowst,
                zbuf, acc):
  c = lax.axis_index("c")
  s = lax.axis_index("s")
  ebase = s * EPT
  bufs = ((colv0, rowv0, wv0, rows0, gsem0, ssem0),
          (colv1, rowv1, wv1, rows1, gsem1, ssem1))

  def zb(j, _):
    zbuf[j, pl.ds(0, Q)] = jnp.zeros((Q,), jnp.float32)
    return 0
  lax.fori_loop(0, ZR, zb, 0, unroll=8)

  def start_chunk(i, q, src, b):
    colv, rowv, wv, rows, gsem, _ = bufs[b]
    base = ebase + i * EC
    pltpu.sync_copy(ec_hbm.at[pl.ds(base, EC)], colv)
    gd = pltpu.async_copy(src.at[q].at[colv], rows, gsem)
    pltpu.sync_copy(er_hbm.at[pl.ds(base, EC)], rowv)
    pltpu.sync_copy(ew_hbm.at[pl.ds(base, EC)], wv)
    return gd

  def work_chunk(gd, b):
    _, rowv, wv, rows, _, ssem = bufs[b]
    gd.wait()

    def scale(j16, _):
      w16 = wv[pl.ds(j16 * 16, 16)]
      for k in range(16):
        ws = jnp.full((16,), w16[k], jnp.float32)
        jj = j16 * 16 + k
        rows[jj, pl.ds(0, Q)] = rows[jj, pl.ds(0, Q)] * ws
      return 0
    lax.fori_loop(0, EC // 16, scale, 0)
    return pltpu.async_copy(rows, acc.at[rowv], ssem, add=True)

  def tail_chunk(q, src):
    base = ebase + NCK * EC
    pltpu.sync_copy(ec_hbm.at[pl.ds(base, ECT)], colvt)
    pltpu.sync_copy(er_hbm.at[pl.ds(base, ECT)], rowvt)
    pltpu.sync_copy(ew_hbm.at[pl.ds(base, ECT)], wvt)
    pltpu.async_copy(src.at[q].at[colvt], rowst, gsem0).wait()

    def scale(j16, _):
      w16 = wvt[pl.ds(j16 * 16, 16)]
      for k in range(16):
        ws = jnp.full((16,), w16[k], jnp.float32)
        jj = j16 * 16 + k
        rowst[jj, pl.ds(0, Q)] = rowst[jj, pl.ds(0, Q)] * ws
      return 0
    lax.fori_loop(0, ECT // 16, scale, 0)
    pltpu.sync_copy(rowst, acc.at[rowvt], add=True)

  for p in range(NG // 2):
    q = c + 2 * p
    for src, dst in ((x_hbm, y1_hbm), (y1_hbm, y2_hbm), (y2_hbm, y3_hbm)):
      for z in range(ZPS):
        cid = s + NS * z

        @pl.when(cid < NCH)
        def _():
          off = pl.multiple_of(cid * ZR, 8)
          pltpu.sync_copy(zbuf, acc.at[pl.ds(off, ZR)])
      plsc.subcore_barrier()

      def chunk2(i2, _):
        g0 = start_chunk(2 * i2, q, src, 0)
        g1 = start_chunk(2 * i2 + 1, q, src, 1)
        s0 = work_chunk(g0, 0)
        s1 = work_chunk(g1, 1)
        s0.wait()
        s1.wait()
        return 0
      lax.fori_loop(0, NCK // 2, chunk2, 0)
      tail_chunk(q, src)

      plsc.subcore_barrier()

      for z in range(ZPS):
        cid = s + NS * z

        @pl.when(cid < NCH)
        def _():
          off = pl.multiple_of(cid * ZR, 8)
          pltpu.sync_copy(acc.at[pl.ds(off, ZR)], dst.at[q].at[pl.ds(off, ZR)])
      plsc.subcore_barrier()


def _finalize_body(a_ref, b_ref, c_ref, d_ref, o_ref):
  for g in range(NG):
    o_ref[:, Q * g:Q * (g + 1)] = 0.25 * (
        a_ref[g] + b_ref[g] + c_ref[g] + d_ref[g])


def _qgather_body(light_hbm, users_hbm, o_hbm, idxv, rowsv, sem):
  wid = lax.axis_index("s") * NC + lax.axis_index("c")
  bpw = 1024 // (NC * NS)
  base = wid * bpw
  pltpu.sync_copy(users_hbm.at[pl.ds(base, bpw)], idxv)
  pltpu.async_copy(light_hbm.at[idxv], rowsv, sem).wait()
  pltpu.sync_copy(rowsv, o_hbm.at[pl.ds(base, bpw)])


def _rating_body(q_ref, it_ref, o_ref):
  s = lax.dot_general(q_ref[...], it_ref[...], (((1,), (1,)), ((), ())),
                      precision=lax.Precision.DEFAULT)
  o_ref[...] = jax.nn.sigmoid(s)


def kernel(users, user_text, item_text, user_image, item_image, user_id_w,
           item_id_w, w1, b1, w2, b2, wi1, bi1, wi2, bi2,
           edge_row, edge_col, edge_w):
  f32 = jnp.float32
  ht = w1.shape[1]
  htp = 256
  w1p = jnp.pad(w1, ((0, 0), (0, htp - ht)))
  b1p = jnp.pad(b1, (0, htp - ht)).reshape(1, htp)
  w2p = jnp.pad(w2, ((0, htp - ht), (0, 0)))
  b2p = b2.reshape(1, LAT)
  hi = wi1.shape[1]
  bi1p = bi1.reshape(1, hi)
  bi2p = bi2.reshape(1, LAT)

  def embed(textA, imgA, idA, n):
    return pl.pallas_call(
        _embed_body,
        grid=(n // RB,),
        in_specs=[
            pl.BlockSpec((RB, textA.shape[1]), lambda i: (i, 0)),
            pl.BlockSpec((RB, imgA.shape[1]), lambda i: (i, 0)),
            pl.BlockSpec((RB, LAT), lambda i: (i, 0)),
            pl.BlockSpec((textA.shape[1], htp), lambda i: (0, 0)),
            pl.BlockSpec((1, htp), lambda i: (0, 0)),
            pl.BlockSpec((htp, LAT), lambda i: (0, 0)),
            pl.BlockSpec((1, LAT), lambda i: (0, 0)),
            pl.BlockSpec((imgA.shape[1], hi), lambda i: (0, 0)),
            pl.BlockSpec((1, hi), lambda i: (0, 0)),
            pl.BlockSpec((hi, LAT), lambda i: (0, 0)),
            pl.BlockSpec((1, LAT), lambda i: (0, 0)),
        ],
        out_specs=pl.BlockSpec((NG, RB, Q), lambda i: (0, i, 0)),
        out_shape=jax.ShapeDtypeStruct((NG, n, Q), f32),
    )(textA, imgA, idA, w1p, b1p, w2p, b2p, wi1, bi1p, wi2, bi2p)

  xu = embed(user_text, user_image, user_id_w, NU)
  xi = embed(item_text, item_image, item_id_w, NI)
  x0 = jnp.concatenate([xu, xi], axis=1)

  mesh = plsc.VectorSubcoreMesh(core_axis_name="c", subcore_axis_name="s",
                                num_cores=NC, num_subcores=NS)
  spmm3 = functools.partial(
      pl.kernel,
      out_type=(jax.ShapeDtypeStruct((NG, NN, Q), f32),) * 3,
      mesh=mesh,
      scratch_types=[
          pltpu.VMEM((EC,), jnp.int32),
          pltpu.VMEM((EC,), jnp.int32),
          pltpu.VMEM((EC,), f32),
          pltpu.VMEM((EC, Q), f32),
          pltpu.SemaphoreType.DMA,
          pltpu.SemaphoreType.DMA,
          pltpu.VMEM((EC,), jnp.int32),
          pltpu.VMEM((EC,), jnp.int32),
          pltpu.VMEM((EC,), f32),
          pltpu.VMEM((EC, Q), f32),
          pltpu.SemaphoreType.DMA,
          pltpu.SemaphoreType.DMA,
          pltpu.VMEM((ECT,), jnp.int32),
          pltpu.VMEM((ECT,), jnp.int32),
          pltpu.VMEM((ECT,), f32),
          pltpu.VMEM((ECT, Q), f32),
          pltpu.VMEM((ZR, Q), f32),
          pltpu.VMEM_SHARED((NN, Q), f32),
      ],
      compiler_params=pltpu.CompilerParams(use_tc_tiling_on_sc=False),
  )(_spmm3_body)

  x1, x2, x3 = spmm3(x0, edge_row, edge_col, edge_w)

  def finalize(n, off):
    return pl.pallas_call(
        _finalize_body,
        grid=(n // RB,),
        in_specs=[pl.BlockSpec((NG, RB, Q), lambda i: (0, i + off, 0))] * 4,
        out_specs=pl.BlockSpec((RB, NG * Q), lambda i: (i, 0)),
        out_shape=jax.ShapeDtypeStruct((n, NG * Q), f32),
    )(x0, x1, x2, x3)

  all_users = finalize(NU, 0)
  all_items = finalize(NI, NU // RB)

  qu = pl.kernel(
      _qgather_body,
      out_type=jax.ShapeDtypeStruct((1024, NG * Q), f32),
      mesh=mesh,
      scratch_types=[
          pltpu.VMEM((1024 // (NC * NS),), jnp.int32),
          pltpu.VMEM((1024 // (NC * NS), NG * Q), f32),
          pltpu.SemaphoreType.DMA,
      ],
  )(all_users, users)

  IB = 512
  rating = pl.pallas_call(
      _rating_body,
      grid=(pl.cdiv(NI, IB),),
      in_specs=[
          pl.BlockSpec((1024, NG * Q), lambda j: (0, 0)),
          pl.BlockSpec((IB, NG * Q), lambda j: (j, 0)),
      ],
      out_specs=pl.BlockSpec((1024, IB), lambda j: (0, j)),
      out_shape=jax.ShapeDtypeStruct((1024, NI), f32),
  )(qu, all_items)

  return (rating, all_users, all_items)
